# Initial kernel scaffold; baseline (speedup 1.0000x reference)
import jax
import jax.numpy as jnp
from jax.experimental import pallas as pl


def kernel(x, edge_index, W1, a_src1, a_dst1, b1, W2, a_src2, a_dst2, b2):
    raise NotImplementedError("write your pallas kernel here")


# speedup vs baseline: 57.2837x; 57.2837x over previous
import functools

import jax
import jax.numpy as jnp
from jax import lax
from jax.experimental import pallas as pl
from jax.experimental.pallas import tpu as pltpu
from jax.experimental.pallas import tpu_sc as plsc

_N = 10000
_E = 320000
_D = 128
_F = 64
_NC = 2
_NS = 16
_K = 80
_EPT = _E // (_NC * _NS)
_NCH = _EPT // _K
_NP = 10240
_RPS = _NP // _NS
_ROWBLK = 2000


def _lrelu_exp(t):
    return jnp.exp(jnp.maximum(t, 0.2 * t))


def _prep1_body(x_ref, w1_ref, as_ref, ad_ref, src_ref, dst_ref, init_ref):
    h = jnp.dot(x_ref[...], w1_ref[...], preferred_element_type=jnp.float32)
    ase = jnp.dot(h, as_ref[...], preferred_element_type=jnp.float32)
    ade = jnp.dot(h, ad_ref[...], preferred_element_type=jnp.float32)
    w_self = _lrelu_exp(ase + ade)
    src_ref[...] = jnp.concatenate([h, ase], axis=1)
    dst_ref[...] = jnp.concatenate([ade, ade], axis=1)
    init_ref[...] = jnp.concatenate([w_self * h, w_self], axis=1)


def _mid_body(part_ref, b1_ref, w2_ref, as_ref, ad_ref,
              src_ref, dst_ref, init_ref):
    p = part_ref[0] + part_ref[1]
    o = p[:, 0:_F] / (p[:, _F:2 * _F] + 1e-16) + b1_ref[0:1, :]
    o = jnp.where(o > 0.0, o, jnp.exp(jnp.minimum(o, 0.0)) - 1.0)
    h2 = jnp.dot(o, w2_ref[...], preferred_element_type=jnp.float32)
    ase = jnp.dot(h2, as_ref[...], preferred_element_type=jnp.float32)
    ade = jnp.dot(h2, ad_ref[...], preferred_element_type=jnp.float32)
    w_self = _lrelu_exp(ase + ade)
    src_ref[...] = jnp.concatenate([h2, ase], axis=1)
    dst_ref[...] = jnp.concatenate([ade, ade], axis=1)
    init_ref[...] = jnp.concatenate([w_self * h2, w_self], axis=1)


def _final_body(part_ref, b2_ref, out_ref):
    p = part_ref[0] + part_ref[1]
    out_ref[...] = p[:, 0:_F] / (p[:, _F:2 * _F] + 1e-16) + b2_ref[0:1, :]


def _row_call(body, extra_shapes, out_widths):
    n_blk = _N // _ROWBLK
    if len(extra_shapes[0]) == 3:
        in_specs = [pl.BlockSpec((2, _ROWBLK, _D), lambda i: (0, i, 0))]
    else:
        in_specs = [pl.BlockSpec((_ROWBLK, _D), lambda i: (i, 0))]
    in_specs += [pl.BlockSpec(s, lambda i, _r=len(s): (0,) * _r)
                 for s in extra_shapes[1:]]
    return pl.pallas_call(
        body,
        grid=(n_blk,),
        in_specs=in_specs,
        out_specs=[pl.BlockSpec((_ROWBLK, w), lambda i: (i, 0))
                   for w in out_widths],
        out_shape=[jax.ShapeDtypeStruct((_N, w), jnp.float32)
                   for w in out_widths],
    )


def _sc_body(src_idx, dst_idx, src_tab, dst_tab, init_tab, out,
             sidx_v, didx_v, srows_v, drows_v, mw_v, acc_sh,
             sem1, sem2):
    c = lax.axis_index("c")
    s = lax.axis_index("s")
    wid = s * _NC + c
    pltpu.sync_copy(init_tab.at[c, pl.ds(s * _RPS, _RPS), :],
                    acc_sh.at[pl.ds(s * _RPS, _RPS), :])
    plsc.subcore_barrier()

    base = wid * _EPT

    def chunk(i, carry):
        off = base + i * _K
        pltpu.sync_copy(src_idx.at[pl.ds(off, _K)], sidx_v)
        pltpu.sync_copy(dst_idx.at[pl.ds(off, _K)], didx_v)
        cp1 = pltpu.async_copy(src_tab.at[sidx_v], srows_v, sem1)
        cp2 = pltpu.async_copy(dst_tab.at[didx_v], drows_v, sem2)
        cp1.wait()
        cp2.wait()

        def edge(k, carry2):
            for j in range(_F // 16):
                sl = pl.ds(16 * j, 16)
                sl_hi = pl.ds(_F + 16 * j, 16)
                w = _lrelu_exp(srows_v[k, sl_hi] + drows_v[k, sl])
                mw_v[k, sl] = srows_v[k, sl] * w
                mw_v[k, sl_hi] = w
            return carry2

        lax.fori_loop(0, _K, edge, 0)
        pltpu.sync_copy(mw_v, acc_sh.at[didx_v], add=True)
        return carry

    lax.fori_loop(0, _NCH, chunk, 0)
    plsc.subcore_barrier()
    pltpu.sync_copy(acc_sh.at[pl.ds(s * _RPS, _RPS), :],
                    out.at[c, pl.ds(s * _RPS, _RPS), :])


@functools.lru_cache(maxsize=1)
def _get_sc_edge_pass():
    mesh = plsc.VectorSubcoreMesh(core_axis_name="c", subcore_axis_name="s")
    return pl.kernel(
        _sc_body,
        out_type=jax.ShapeDtypeStruct((_NC, _NP, _D), jnp.float32),
        mesh=mesh,
        scratch_types=[
            pltpu.VMEM((_K,), jnp.int32),
            pltpu.VMEM((_K,), jnp.int32),
            pltpu.VMEM((_K, _D), jnp.float32),
            pltpu.VMEM((_K, _D), jnp.float32),
            pltpu.VMEM((_K, _D), jnp.float32),
            pltpu.VMEM_SHARED((_NP, _D), jnp.float32),
            pltpu.SemaphoreType.DMA,
            pltpu.SemaphoreType.DMA,
        ],
    )


def _sc_edge_pass(src_ids, dst_ids, src_tab, dst_tab, init_full):
    pad = jnp.zeros((_NC, _NP - _N, _D), jnp.float32)
    init_pad = jnp.concatenate([init_full, pad], axis=1)
    part = _get_sc_edge_pass()(src_ids, dst_ids, src_tab, dst_tab, init_pad)
    return part[:, :_N, :]


def _expand_coeff1(a):
    eye8 = jnp.eye(8, dtype=jnp.float32)
    full = a[:, :, None, None] * eye8[:, None, :, None]
    return jnp.broadcast_to(full, (8, 8, 8, 8)).reshape(64, 64)


def _expand_coeff2(a):
    return jnp.broadcast_to(a.reshape(_F, 1), (_F, _F))


def kernel(x, edge_index, W1, a_src1, a_dst1, b1, W2, a_src2, a_dst2, b2):
    src_ids = edge_index[0]
    dst_ids = edge_index[1]

    prep1 = _row_call(_prep1_body,
                      [(_ROWBLK, _D), (_D, _F), (_F, _F), (_F, _F)],
                      [_D, _D, _D])
    src1, dst1, init1 = prep1(x, W1, _expand_coeff1(a_src1),
                              _expand_coeff1(a_dst1))
    init1_full = jnp.stack([init1, jnp.zeros_like(init1)])
    part1 = _sc_edge_pass(src_ids, dst_ids, src1, dst1, init1_full)

    b1_pad = jnp.broadcast_to(b1.reshape(1, _F), (8, _F))
    b2_pad = jnp.broadcast_to(b2.reshape(1, _F), (8, _F))

    mid = _row_call(_mid_body,
                    [(2, _ROWBLK, _D), (8, _F), (_F, _F), (_F, _F), (_F, _F)],
                    [_D, _D, _D])
    src2, dst2, init2 = mid(part1, b1_pad, W2,
                            _expand_coeff2(a_src2), _expand_coeff2(a_dst2))
    init2_full = jnp.stack([init2, jnp.zeros_like(init2)])
    part2 = _sc_edge_pass(src_ids, dst_ids, src2, dst2, init2_full)

    final = _row_call(_final_body, [(2, _ROWBLK, _D), (8, _F)], [_F])
    (out,) = final(part2, b2_pad)
    return out


# speedup vs baseline: 99.7944x; 1.7421x over previous
import functools

import jax
import jax.numpy as jnp
from jax import lax
from jax.experimental import pallas as pl
from jax.experimental.pallas import tpu as pltpu
from jax.experimental.pallas import tpu_sc as plsc

_N = 10000
_E = 320000
_D = 128
_F = 64
_NC = 2
_NS = 16
_K = 40
_EPT = _E // (_NC * _NS)
_NCH = _EPT // _K
_SB = 50
_NSB = _NCH // _SB
_NP = 10240
_RPS = _NP // _NS
_ROWBLK = 2000


def _lrelu_exp(t):
    return jnp.exp(jnp.maximum(t, 0.2 * t))


def _prep1_body(x_ref, w1_ref, as_ref, ad_ref, src_ref, dst_ref, init_ref):
    h = jnp.dot(x_ref[...], w1_ref[...], preferred_element_type=jnp.float32)
    ase = jnp.dot(h, as_ref[...], preferred_element_type=jnp.float32)
    ade = jnp.dot(h, ad_ref[...], preferred_element_type=jnp.float32)
    w_self = _lrelu_exp(ase + ade)
    src_ref[...] = jnp.concatenate([h, ase], axis=1)
    dst_ref[...] = jnp.concatenate([ade, ade], axis=1)
    init_ref[...] = jnp.concatenate([w_self * h, w_self], axis=1)


def _mid_body(part_ref, b1_ref, w2_ref, as_ref, ad_ref,
              src_ref, dst_ref, init_ref):
    p = part_ref[0] + part_ref[1]
    o = p[:, 0:_F] / (p[:, _F:2 * _F] + 1e-16) + b1_ref[0:1, :]
    o = jnp.where(o > 0.0, o, jnp.exp(jnp.minimum(o, 0.0)) - 1.0)
    h2 = jnp.dot(o, w2_ref[...], preferred_element_type=jnp.float32)
    ase = jnp.dot(h2, as_ref[...], preferred_element_type=jnp.float32)
    ade = jnp.dot(h2, ad_ref[...], preferred_element_type=jnp.float32)
    w_self = _lrelu_exp(ase + ade)
    src_ref[...] = jnp.concatenate([h2, ase], axis=1)
    dst_ref[...] = jnp.concatenate([ade, ade], axis=1)
    init_ref[...] = jnp.concatenate([w_self * h2, w_self], axis=1)


def _final_body(part_ref, b2_ref, out_ref):
    p = part_ref[0] + part_ref[1]
    out_ref[...] = p[:, 0:_F] / (p[:, _F:2 * _F] + 1e-16) + b2_ref[0:1, :]


def _row_call(body, extra_shapes, out_widths):
    n_blk = _N // _ROWBLK
    if len(extra_shapes[0]) == 3:
        in_specs = [pl.BlockSpec((2, _ROWBLK, _D), lambda i: (0, i, 0))]
    else:
        in_specs = [pl.BlockSpec((_ROWBLK, _D), lambda i: (i, 0))]
    in_specs += [pl.BlockSpec(s, lambda i, _r=len(s): (0,) * _r)
                 for s in extra_shapes[1:]]
    return pl.pallas_call(
        body,
        grid=(n_blk,),
        in_specs=in_specs,
        out_specs=[pl.BlockSpec((_ROWBLK, w), lambda i: (i, 0))
                   for w in out_widths],
        out_shape=[jax.ShapeDtypeStruct((_N, w), jnp.float32)
                   for w in out_widths],
    )


def _sc_body(src_idx, dst_idx, src_tab, dst_tab, init_tab, out,
             sidx_blk, didx_blk,
             srows_a, drows_a, mw_a, srows_b, drows_b, mw_b, acc_sh,
             g1a, g2a, sa, g1b, g2b, sb_sem):
    c = lax.axis_index("c")
    s = lax.axis_index("s")
    wid = s * _NC + c
    pltpu.sync_copy(init_tab.at[c, pl.ds(s * _RPS, _RPS), :],
                    acc_sh.at[pl.ds(s * _RPS, _RPS), :])
    plsc.subcore_barrier()

    def issue_gather(li, srows_v, drows_v, g1, g2):
        pltpu.async_copy(src_tab.at[sidx_blk.at[li]], srows_v, g1)
        pltpu.async_copy(dst_tab.at[didx_blk.at[li]], drows_v, g2)

    def wait_gather(srows_v, drows_v, g1, g2):
        pltpu.make_async_copy(src_tab.at[pl.ds(0, _K), :], srows_v, g1).wait()
        pltpu.make_async_copy(dst_tab.at[pl.ds(0, _K), :], drows_v, g2).wait()

    def prime_scatter(mw_v, ssem):
        pltpu.async_copy(init_tab.at[0, pl.ds(0, _K), :], mw_v, ssem)

    def wait_scatter(mw_v, ssem):
        pltpu.make_async_copy(init_tab.at[0, pl.ds(0, _K), :], mw_v,
                              ssem).wait()

    def compute(srows_v, drows_v, mw_v):
        def edge(k, carry2):
            for j in range(_F // 16):
                sl = pl.ds(16 * j, 16)
                sl_hi = pl.ds(_F + 16 * j, 16)
                w = _lrelu_exp(srows_v[k, sl_hi] + drows_v[k, sl])
                mw_v[k, sl] = srows_v[k, sl] * w
                mw_v[k, sl_hi] = w
            return carry2

        lax.fori_loop(0, _K, edge, 0)

    prime_scatter(mw_a, sa)
    prime_scatter(mw_b, sb_sem)

    def superblock(sbi, carry):
        wait_scatter(mw_a, sa)
        wait_scatter(mw_b, sb_sem)
        pltpu.sync_copy(src_idx.at[wid, sbi], sidx_blk)
        pltpu.sync_copy(dst_idx.at[wid, sbi], didx_blk)
        prime_scatter(mw_a, sa)
        prime_scatter(mw_b, sb_sem)
        issue_gather(0, srows_a, drows_a, g1a, g2a)
        issue_gather(1, srows_b, drows_b, g1b, g2b)

        def body(i2, carry2):
            l0 = 2 * i2
            l1 = l0 + 1
            wait_gather(srows_a, drows_a, g1a, g2a)
            wait_scatter(mw_a, sa)
            compute(srows_a, drows_a, mw_a)
            pltpu.async_copy(mw_a, acc_sh.at[didx_blk.at[l0]], sa, add=True)
            issue_gather(l0 + 2, srows_a, drows_a, g1a, g2a)
            wait_gather(srows_b, drows_b, g1b, g2b)
            wait_scatter(mw_b, sb_sem)
            compute(srows_b, drows_b, mw_b)
            pltpu.async_copy(mw_b, acc_sh.at[didx_blk.at[l1]], sb_sem,
                             add=True)
            issue_gather(l1 + 2, srows_b, drows_b, g1b, g2b)
            return carry2

        lax.fori_loop(0, _SB // 2 - 1, body, 0)

        wait_gather(srows_a, drows_a, g1a, g2a)
        wait_scatter(mw_a, sa)
        compute(srows_a, drows_a, mw_a)
        pltpu.async_copy(mw_a, acc_sh.at[didx_blk.at[_SB - 2]], sa, add=True)
        wait_gather(srows_b, drows_b, g1b, g2b)
        wait_scatter(mw_b, sb_sem)
        compute(srows_b, drows_b, mw_b)
        pltpu.async_copy(mw_b, acc_sh.at[didx_blk.at[_SB - 1]], sb_sem,
                         add=True)
        return carry

    lax.fori_loop(0, _NSB, superblock, 0)
    wait_scatter(mw_a, sa)
    wait_scatter(mw_b, sb_sem)

    plsc.subcore_barrier()
    pltpu.sync_copy(acc_sh.at[pl.ds(s * _RPS, _RPS), :],
                    out.at[c, pl.ds(s * _RPS, _RPS), :])


@functools.lru_cache(maxsize=1)
def _get_sc_edge_pass():
    mesh = plsc.VectorSubcoreMesh(core_axis_name="c", subcore_axis_name="s")
    return pl.kernel(
        _sc_body,
        out_type=jax.ShapeDtypeStruct((_NC, _NP, _D), jnp.float32),
        mesh=mesh,
        scratch_types=[
            pltpu.VMEM((_SB, _K), jnp.int32),
            pltpu.VMEM((_SB, _K), jnp.int32),
            pltpu.VMEM((_K, _D), jnp.float32),
            pltpu.VMEM((_K, _D), jnp.float32),
            pltpu.VMEM((_K, _D), jnp.float32),
            pltpu.VMEM((_K, _D), jnp.float32),
            pltpu.VMEM((_K, _D), jnp.float32),
            pltpu.VMEM((_K, _D), jnp.float32),
            pltpu.VMEM_SHARED((_NP, _D), jnp.float32),
            pltpu.SemaphoreType.DMA,
            pltpu.SemaphoreType.DMA,
            pltpu.SemaphoreType.DMA,
            pltpu.SemaphoreType.DMA,
            pltpu.SemaphoreType.DMA,
            pltpu.SemaphoreType.DMA,
        ],
    )


def _sc_edge_pass(src_ids, dst_ids, src_tab, dst_tab, init_full):
    pad = jnp.zeros((_NC, _NP - _N, _D), jnp.float32)
    init_pad = jnp.concatenate([init_full, pad], axis=1)
    nw = _NC * _NS
    part = _get_sc_edge_pass()(src_ids.reshape(nw, _NSB, _SB, _K),
                               dst_ids.reshape(nw, _NSB, _SB, _K),
                               src_tab, dst_tab, init_pad)
    return part[:, :_N, :]


def _expand_coeff1(a):
    eye8 = jnp.eye(8, dtype=jnp.float32)
    full = a[:, :, None, None] * eye8[:, None, :, None]
    return jnp.broadcast_to(full, (8, 8, 8, 8)).reshape(64, 64)


def _expand_coeff2(a):
    return jnp.broadcast_to(a.reshape(_F, 1), (_F, _F))


def kernel(x, edge_index, W1, a_src1, a_dst1, b1, W2, a_src2, a_dst2, b2):
    src_ids = edge_index[0]
    dst_ids = edge_index[1]

    prep1 = _row_call(_prep1_body,
                      [(_ROWBLK, _D), (_D, _F), (_F, _F), (_F, _F)],
                      [_D, _D, _D])
    src1, dst1, init1 = prep1(x, W1, _expand_coeff1(a_src1),
                              _expand_coeff1(a_dst1))
    init1_full = jnp.stack([init1, jnp.zeros_like(init1)])
    part1 = _sc_edge_pass(src_ids, dst_ids, src1, dst1, init1_full)

    b1_pad = jnp.broadcast_to(b1.reshape(1, _F), (8, _F))
    b2_pad = jnp.broadcast_to(b2.reshape(1, _F), (8, _F))

    mid = _row_call(_mid_body,
                    [(2, _ROWBLK, _D), (8, _F), (_F, _F), (_F, _F), (_F, _F)],
                    [_D, _D, _D])
    src2, dst2, init2 = mid(part1, b1_pad, W2,
                            _expand_coeff2(a_src2), _expand_coeff2(a_dst2))
    init2_full = jnp.stack([init2, jnp.zeros_like(init2)])
    part2 = _sc_edge_pass(src_ids, dst_ids, src2, dst2, init2_full)

    final = _row_call(_final_body, [(2, _ROWBLK, _D), (8, _F)], [_F])
    (out,) = final(part2, b2_pad)
    return out
